# baseline (speedup 1.0000x reference)
import functools

import jax
import jax.numpy as jnp
from jax import lax
from jax.experimental import pallas as pl
from jax.experimental.pallas import tpu as pltpu
from jax.experimental.pallas import tpu_sc as plsc

_CHUNK = 128


def kernel(items, tf_matrix):
    (batch,) = items.shape
    _, dim = tf_matrix.shape

    info = plsc.get_sparse_core_info()
    num_workers = info.num_cores * info.num_subcores
    b_per_w = batch // num_workers
    n_chunks = b_per_w // _CHUNK

    idx3 = items.astype(jnp.int32).reshape(num_workers, n_chunks, _CHUNK)

    mesh = plsc.VectorSubcoreMesh(core_axis_name="c", subcore_axis_name="s")

    @functools.partial(
        pl.kernel,
        mesh=mesh,
        out_type=jax.ShapeDtypeStruct((batch, dim), tf_matrix.dtype),
        scratch_types=[
            pltpu.VMEM((n_chunks, _CHUNK), jnp.int32),
            pltpu.VMEM((b_per_w, dim), jnp.float32),
            pltpu.SemaphoreType.DMA,
        ],
        compiler_params=pltpu.CompilerParams(use_tc_tiling_on_sc=False),
    )
    def gather_kernel(items_hbm, table_hbm, out_hbm, idx_v, rows_v, sem):
        wid = lax.axis_index("s") * info.num_cores + lax.axis_index("c")
        base = wid * b_per_w
        pltpu.sync_copy(items_hbm.at[wid], idx_v)
        copies = [
            pltpu.async_copy(
                table_hbm.at[idx_v.at[j]],
                rows_v.at[pl.ds(j * _CHUNK, _CHUNK)],
                sem,
            )
            for j in range(n_chunks)
        ]
        for c in copies:
            c.wait()
        pltpu.sync_copy(rows_v, out_hbm.at[pl.ds(base, b_per_w)])

    return gather_kernel(idx3, tf_matrix)


# speedup vs baseline: 3.6569x; 3.6569x over previous
import functools

import jax
import jax.numpy as jnp
from jax import lax
from jax.experimental import pallas as pl
from jax.experimental.pallas import tpu as pltpu
from jax.experimental.pallas import tpu_sc as plsc

_L = 16
_GRP = 8
_BLK = 128
_PHASE = 128


def kernel(items, tf_matrix):
    (batch,) = items.shape
    vocab, dim = tf_matrix.shape
    assert dim == 32

    n_full_blocks = vocab // _BLK
    tail_base = n_full_blocks * _BLK

    table_t = tf_matrix.T
    tail_t = jnp.pad(table_t[:, tail_base:], ((0, 0), (0, _BLK - (vocab - tail_base))))

    idx8 = items.astype(jnp.int32).reshape(batch // _GRP, _GRP)
    idx_sp = jnp.pad(idx8, ((0, 0), (0, _L - _GRP))).reshape(-1)

    info = plsc.get_sparse_core_info()
    num_workers = info.num_cores * info.num_subcores
    b_per_w = batch // num_workers
    n_phases = b_per_w // _PHASE
    grp_per_phase = _PHASE // _GRP

    mesh = plsc.VectorSubcoreMesh(core_axis_name="c", subcore_axis_name="s")

    scratch = (
        [pltpu.VMEM((b_per_w * 2,), jnp.int32)]
        + [pltpu.VMEM((dim, _BLK), jnp.float32) for _ in range(2 * _GRP)]
        + [pltpu.VMEM((dim, _BLK), jnp.float32) for _ in range(2)]
        + [pltpu.VMEM((dim, _BLK), jnp.float32)]
        + [pltpu.SemaphoreType.DMA for _ in range(5)]
    )

    @functools.partial(
        pl.kernel,
        mesh=mesh,
        out_type=jax.ShapeDtypeStruct((dim, batch), jnp.float32),
        scratch_types=scratch,
        compiler_params=pltpu.CompilerParams(
            use_tc_tiling_on_sc=True, needs_layout_passes=False
        ),
    )
    def gather_kernel(idx_hbm, table_hbm, tail_hbm, out_hbm, idx_v, *rest):
        slots = rest[: 2 * _GRP]
        cbs = rest[2 * _GRP : 2 * _GRP + 2]
        tail_v = rest[2 * _GRP + 2]
        sem_a, sem_b, sem_cb0, sem_cb1, sem_tail = rest[2 * _GRP + 3 :]
        half_sems = (sem_a, sem_b)
        cb_sems = (sem_cb0, sem_cb1)

        wid = lax.axis_index("s") * info.num_cores + lax.axis_index("c")
        pltpu.sync_copy(idx_hbm.at[pl.ds(wid * b_per_w * 2, b_per_w * 2)], idx_v)
        pltpu.async_copy(tail_hbm, tail_v, sem_tail).wait()

        iota = lax.iota(jnp.int32, _L)

        def lane_info(vec, b):
            v = vec[b]
            c = lax.shift_right_logical(v, 7)
            start = pl.multiple_of(c * _BLK, _BLK)
            return v, start, c < n_full_blocks

        def issue_group(g, half):
            vec = idx_v[pl.ds(g * _L, _L)]
            for b in range(_GRP):
                v, start, main = lane_info(vec, b)

                @pl.when(main)
                def _():
                    pltpu.async_copy(
                        table_hbm.at[:, pl.ds(start, _BLK)],
                        slots[half * _GRP + b],
                        half_sems[half],
                    )

        def drain_extract(g, half, cb):
            vec = idx_v[pl.ds(g * _L, _L)]
            for b in range(_GRP):
                v, start, main = lane_info(vec, b)

                @pl.when(main)
                def _():
                    pltpu.make_async_copy(
                        table_hbm.at[:, pl.ds(start, _BLK)],
                        slots[half * _GRP + b],
                        half_sems[half],
                    ).wait()

            for b in range(_GRP):
                v, start, main = lane_info(vec, b)
                n_loc = (g % grp_per_phase) * _GRP + b
                dst_col = jnp.full((_L,), n_loc, jnp.int32)

                @pl.when(main)
                def _():
                    u = jnp.full((_L,), v & (_BLK - 1), jnp.int32)
                    src = slots[half * _GRP + b]
                    x0 = plsc.load_gather(src, [iota, u])
                    x1 = plsc.load_gather(src, [iota + _L, u])
                    plsc.store_scatter(cb, [iota, dst_col], x0)
                    plsc.store_scatter(cb, [iota + _L, dst_col], x1)

                @pl.when(jnp.logical_not(main))
                def _():
                    ut = jnp.full((_L,), v - tail_base, jnp.int32)
                    x0 = plsc.load_gather(tail_v, [iota, ut])
                    x1 = plsc.load_gather(tail_v, [iota + _L, ut])
                    plsc.store_scatter(cb, [iota, dst_col], x0)
                    plsc.store_scatter(cb, [iota + _L, dst_col], x1)

        def out_win(p):
            col = pl.multiple_of(wid * b_per_w + p * _PHASE, _BLK)
            return out_hbm.at[:, pl.ds(col, _PHASE)]

        for p in range(n_phases):
            cb = cbs[p % 2]
            sem_cb = cb_sems[p % 2]
            if p >= 2:
                pltpu.make_async_copy(cb, out_win(p - 2), sem_cb).wait()
            g0 = p * grp_per_phase
            issue_group(g0, 0)

            def body(k, _):
                ga = g0 + 2 * k
                issue_group(ga + 1, 1)
                drain_extract(ga, 0, cb)

                @pl.when(2 * k + 2 < grp_per_phase)
                def _():
                    issue_group(ga + 2, 0)

                drain_extract(ga + 1, 1, cb)
                return 0

            lax.fori_loop(0, grp_per_phase // 2, body, 0)
            pltpu.async_copy(cb, out_win(p), sem_cb)

        pltpu.make_async_copy(cbs[0], out_win(n_phases - 2), sem_cb0).wait()
        pltpu.make_async_copy(cbs[1], out_win(n_phases - 1), sem_cb1).wait()

    return gather_kernel(idx_sp, table_t, tail_t).T
